# Initial kernel scaffold; baseline (speedup 1.0000x reference)
import functools

import jax
import jax.numpy as jnp
from jax import lax
from jax.experimental import pallas as pl
from jax.experimental.pallas import tpu as pltpu
from jax.experimental.pallas import tpu_sc as plsc

N = 10000
C = 128
NS = 16
NT = N * NS
PD = 16


_RQ = 2000


def _qkv_body(f_ref, w_ref, b_ref, q_ref, k_ref, v_ref):
    y = jnp.dot(f_ref[...], w_ref[...], preferred_element_type=jnp.float32) + b_ref[...]
    q_ref[...] = y[:, :C]
    k_ref[...] = y[:, C:2 * C]
    v_ref[...] = y[:, 2 * C:]


def _qkv(features, wqkv, bqkv):
    return pl.pallas_call(
        _qkv_body,
        grid=(N // _RQ,),
        in_specs=[pl.BlockSpec((_RQ, C), lambda i: (i, 0)),
                  pl.BlockSpec((C, 3 * C), lambda i: (0, 0)),
                  pl.BlockSpec((1, 3 * C), lambda i: (0, 0))],
        out_specs=[pl.BlockSpec((_RQ, C), lambda i: (i, 0))] * 3,
        out_shape=[jax.ShapeDtypeStruct((N, C), jnp.float32)] * 3,
    )(features, wqkv, bqkv)


_QB = 80


def _knn_body(q_ref, p_ref, idx_ref):
    qx = q_ref[:, 0:1]
    qy = q_ref[:, 1:2]
    qz = q_ref[:, 2:3]
    px = p_ref[0:1, :]
    py = p_ref[1:2, :]
    pz = p_ref[2:3, :]
    dx = qx - px
    dy = qy - py
    dz = qz - pz
    d2 = dx * dx + dy * dy + dz * dz
    ii = lax.broadcasted_iota(jnp.int32, (_QB, N), 1)
    big = jnp.int32(2 ** 30)
    for t in range(NS):
        m = jnp.min(d2, axis=1, keepdims=True)
        cand = jnp.where(d2 <= m, ii, big)
        j = jnp.min(cand, axis=1, keepdims=True)
        idx_ref[:, t:t + 1] = j
        d2 = jnp.where(ii == j, jnp.float32(jnp.inf), d2)


def _knn(pts_pad, pts8):
    return pl.pallas_call(
        _knn_body,
        grid=(N // _QB,),
        in_specs=[pl.BlockSpec((_QB, PD), lambda i: (i, 0)),
                  pl.BlockSpec((8, N), lambda i: (0, 0))],
        out_specs=pl.BlockSpec((_QB, NS), lambda i: (i, 0)),
        out_shape=jax.ShapeDtypeStruct((N, NS), jnp.int32),
    )(pts_pad, pts8)


_NC = 2
_NSUB = 16
_NW = _NC * _NSUB
_BPW = NT // _NW
_CH = 128
_NCH = 40
_TAIL = _BPW - _CH


def _sc_gather(k, v, pts_pad, idx_flat):
    mesh = plsc.VectorSubcoreMesh(core_axis_name="c", subcore_axis_name="s")

    @functools.partial(
        pl.kernel,
        mesh=mesh,
        out_type=[jax.ShapeDtypeStruct((NT, C), jnp.float32),
                  jax.ShapeDtypeStruct((NT, C), jnp.float32),
                  jax.ShapeDtypeStruct((NT, PD), jnp.float32)],
        scratch_types=[pltpu.VMEM((_CH,), jnp.int32),
                       pltpu.VMEM((_CH, C), jnp.float32),
                       pltpu.VMEM((_CH, C), jnp.float32),
                       pltpu.VMEM((_CH, PD), jnp.float32),
                       pltpu.SemaphoreType.DMA],
    )
    def body(k_hbm, v_hbm, p_hbm, idx_hbm, gk_hbm, gv_hbm, gp_hbm,
             idx_v, kbuf, vbuf, pbuf, sem):
        wid = lax.axis_index("s") * _NC + lax.axis_index("c")
        base = wid * _BPW

        def step(c, carry):
            off = base + jnp.minimum(c * _CH, _TAIL)
            pltpu.sync_copy(idx_hbm.at[pl.ds(off, _CH)], idx_v)
            h1 = pltpu.async_copy(k_hbm.at[idx_v], kbuf, sem)
            h2 = pltpu.async_copy(v_hbm.at[idx_v], vbuf, sem)
            h3 = pltpu.async_copy(p_hbm.at[idx_v], pbuf, sem)
            h1.wait()
            h2.wait()
            h3.wait()
            pltpu.sync_copy(kbuf, gk_hbm.at[pl.ds(off, _CH)])
            pltpu.sync_copy(vbuf, gv_hbm.at[pl.ds(off, _CH)])
            pltpu.sync_copy(pbuf, gp_hbm.at[pl.ds(off, _CH)])
            return carry

        lax.fori_loop(0, _NCH, step, 0)

    return body(k, v, pts_pad, idx_flat)


_RB = 200
_RTB = _RB * NS


def _momA_body(gp_ref, pts_ref, m1_ref, m2_ref):
    i = pl.program_id(0)
    rel = (gp_ref[...] - pts_ref[...][:, None, :]).reshape(_RTB, PD)
    m1 = jnp.sum(rel, axis=0, keepdims=True)
    m2 = lax.dot_general(rel, rel, (((0,), (0,)), ((), ())),
                         preferred_element_type=jnp.float32)

    @pl.when(i == 0)
    def _():
        m1_ref[...] = jnp.zeros_like(m1_ref)
        m2_ref[...] = jnp.zeros_like(m2_ref)

    m1_ref[...] += m1
    m2_ref[...] += m2


def _momA(gp3, pts_pad):
    return pl.pallas_call(
        _momA_body,
        grid=(N // _RB,),
        in_specs=[pl.BlockSpec((_RB, NS, PD), lambda i: (i, 0, 0)),
                  pl.BlockSpec((_RB, PD), lambda i: (i, 0))],
        out_specs=[pl.BlockSpec((1, PD), lambda i: (0, 0)),
                   pl.BlockSpec((PD, PD), lambda i: (0, 0))],
        out_shape=[jax.ShapeDtypeStruct((1, PD), jnp.float32),
                   jax.ShapeDtypeStruct((PD, PD), jnp.float32)],
    )(gp3, pts_pad)


def _passB_body(gp_ref, pts_ref, q_ref, gk_ref, wp_ref, bp_ref, s1_ref, t1_ref,
                wg1_ref, bg1_ref, a1_ref, st_ref):
    i = pl.program_id(0)
    rel = (gp_ref[...] - pts_ref[...][:, None, :]).reshape(_RTB, PD)
    pre = jnp.dot(rel, wp_ref[...], preferred_element_type=jnp.float32) + bp_ref[...]
    p = jnp.maximum(pre * s1_ref[...] + t1_ref[...], 0.0)
    x3 = q_ref[...] - gk_ref[...] + p.reshape(_RB, NS, C)
    a1 = jnp.dot(x3.reshape(_RTB, C), wg1_ref[...],
                 preferred_element_type=jnp.float32) + bg1_ref[...]
    a1_ref[...] = a1

    @pl.when(i == 0)
    def _():
        st_ref[...] = jnp.zeros_like(st_ref)

    st_ref[0:1, :] += jnp.sum(a1, axis=0, keepdims=True)
    st_ref[1:2, :] += jnp.sum(a1 * a1, axis=0, keepdims=True)


def _passB(gp3, pts_pad, q3, gk3, wp, bp, s1, t1, wg1, bg1):
    return pl.pallas_call(
        _passB_body,
        grid=(N // _RB,),
        in_specs=[pl.BlockSpec((_RB, NS, PD), lambda i: (i, 0, 0)),
                  pl.BlockSpec((_RB, PD), lambda i: (i, 0)),
                  pl.BlockSpec((_RB, 1, C), lambda i: (i, 0, 0)),
                  pl.BlockSpec((_RB, NS, C), lambda i: (i, 0, 0)),
                  pl.BlockSpec((PD, C), lambda i: (0, 0)),
                  pl.BlockSpec((1, C), lambda i: (0, 0)),
                  pl.BlockSpec((1, C), lambda i: (0, 0)),
                  pl.BlockSpec((1, C), lambda i: (0, 0)),
                  pl.BlockSpec((C, C), lambda i: (0, 0)),
                  pl.BlockSpec((1, C), lambda i: (0, 0))],
        out_specs=[pl.BlockSpec((_RTB, C), lambda i: (i, 0)),
                   pl.BlockSpec((2, C), lambda i: (0, 0))],
        out_shape=[jax.ShapeDtypeStruct((NT, C), jnp.float32),
                   jax.ShapeDtypeStruct((2, C), jnp.float32)],
    )(gp3, pts_pad, q3, gk3, wp, bp, s1, t1, wg1, bg1)


def _passC_body(gp_ref, pts_ref, gv_ref, a1_ref, wp_ref, bp_ref, s1_ref, t1_ref,
                s2_ref, t2_ref, wg2_ref, bg2_ref, wo_ref, bo_ref, out_ref):
    rel = (gp_ref[...] - pts_ref[...][:, None, :]).reshape(_RTB, PD)
    pre = jnp.dot(rel, wp_ref[...], preferred_element_type=jnp.float32) + bp_ref[...]
    p = jnp.maximum(pre * s1_ref[...] + t1_ref[...], 0.0)
    h = jnp.maximum(a1_ref[...] * s2_ref[...] + t2_ref[...], 0.0)
    a = jnp.dot(h, wg2_ref[...], preferred_element_type=jnp.float32) + bg2_ref[...]
    a3 = a.reshape(_RB, NS, C)
    mx = jnp.max(a3, axis=1, keepdims=True)
    e = jnp.exp(a3 - mx)
    sm = e / jnp.sum(e, axis=1, keepdims=True)
    gvp = gv_ref[...] + p.reshape(_RB, NS, C)
    o = jnp.sum(gvp * sm, axis=1)
    out_ref[...] = jnp.dot(o, wo_ref[...], preferred_element_type=jnp.float32) + bo_ref[...]


def _passC(gp3, pts_pad, gv3, a1, wp, bp, s1, t1, s2, t2, wg2, bg2, wo, bo):
    return pl.pallas_call(
        _passC_body,
        grid=(N // _RB,),
        in_specs=[pl.BlockSpec((_RB, NS, PD), lambda i: (i, 0, 0)),
                  pl.BlockSpec((_RB, PD), lambda i: (i, 0)),
                  pl.BlockSpec((_RB, NS, C), lambda i: (i, 0, 0)),
                  pl.BlockSpec((_RTB, C), lambda i: (i, 0)),
                  pl.BlockSpec((PD, C), lambda i: (0, 0)),
                  pl.BlockSpec((1, C), lambda i: (0, 0)),
                  pl.BlockSpec((1, C), lambda i: (0, 0)),
                  pl.BlockSpec((1, C), lambda i: (0, 0)),
                  pl.BlockSpec((1, C), lambda i: (0, 0)),
                  pl.BlockSpec((1, C), lambda i: (0, 0)),
                  pl.BlockSpec((C, C), lambda i: (0, 0)),
                  pl.BlockSpec((1, C), lambda i: (0, 0)),
                  pl.BlockSpec((C, C), lambda i: (0, 0)),
                  pl.BlockSpec((1, C), lambda i: (0, 0))],
        out_specs=pl.BlockSpec((_RB, C), lambda i: (i, 0)),
        out_shape=jax.ShapeDtypeStruct((N, C), jnp.float32),
    )(gp3, pts_pad, gv3, a1, wp, bp, s1, t1, s2, t2, wg2, bg2, wo, bo)


def kernel(points, features, Wq, bq, Wk, bk, Wv, bv, Wp, bp, g_p, be_p,
           Wg1, bg1, g_g, be_g, Wg2, bg2, Wo, bo):
    pts_pad = jnp.pad(points, ((0, 0), (0, PD - 3)))
    pts8 = jnp.pad(points.T, ((0, 5), (0, 0)))
    wqkv = jnp.concatenate([Wq, Wk, Wv], axis=1)
    bqkv = jnp.concatenate([bq, bk, bv])[None, :]
    wp_pad = jnp.pad(Wp, ((0, PD - 3), (0, 0)))

    q, k, v = _qkv(features, wqkv, bqkv)
    idx = _knn(pts_pad, pts8)
    gk, gv, gp = _sc_gather(k, v, pts_pad, idx.reshape(NT))

    gp3 = gp.reshape(N, NS, PD)
    gk3 = gk.reshape(N, NS, C)
    gv3 = gv.reshape(N, NS, C)
    q3 = q.reshape(N, 1, C)

    m1, m2 = _momA(gp3, pts_pad)

    inv = jnp.float32(1.0 / NT)
    a_mean = (m1[0] * inv) @ wp_pad
    quad = jnp.sum(wp_pad * (m2 @ wp_pad), axis=0) * inv
    mean_p = a_mean + bp
    var_p = quad - a_mean * a_mean
    scale1 = g_p * lax.rsqrt(var_p + 1e-5)
    shift1 = be_p - mean_p * scale1

    a1, st = _passB(gp3, pts_pad, q3, gk3, wp_pad, bp[None, :],
                    scale1[None, :], shift1[None, :], Wg1, bg1[None, :])

    mu_g = st[0] * inv
    var_g = st[1] * inv - mu_g * mu_g
    scale2 = g_g * lax.rsqrt(var_g + 1e-5)
    shift2 = be_g - mu_g * scale2

    return _passC(gp3, pts_pad, gv3, a1, wp_pad, bp[None, :],
                  scale1[None, :], shift1[None, :],
                  scale2[None, :], shift2[None, :],
                  Wg2, bg2[None, :], Wo, bo[None, :])


# speedup vs baseline: 2.5128x; 2.5128x over previous
import functools

import jax
import jax.numpy as jnp
from jax import lax
from jax.experimental import pallas as pl
from jax.experimental.pallas import tpu as pltpu
from jax.experimental.pallas import tpu_sc as plsc

N = 10000
C = 128
NS = 16
NT = N * NS


_RQ = 2000


def _qkv_body(f_ref, w_ref, b_ref, q_ref, k_ref, v_ref):
    y = jnp.dot(f_ref[...], w_ref[...], preferred_element_type=jnp.float32) + b_ref[...]
    q_ref[...] = y[:, :C]
    k_ref[...] = y[:, C:2 * C]
    v_ref[...] = y[:, 2 * C:]


def _qkv(features, wqkv, bqkv):
    return pl.pallas_call(
        _qkv_body,
        grid=(N // _RQ,),
        in_specs=[pl.BlockSpec((_RQ, C), lambda i: (i, 0)),
                  pl.BlockSpec((C, 3 * C), lambda i: (0, 0)),
                  pl.BlockSpec((1, 3 * C), lambda i: (0, 0))],
        out_specs=[pl.BlockSpec((_RQ, C), lambda i: (i, 0))] * 3,
        out_shape=[jax.ShapeDtypeStruct((N, C), jnp.float32)] * 3,
    )(features, wqkv, bqkv)


_QB = 80


def _knn_body(q_ref, p_ref, idx_ref, d2_ref):
    qf = q_ref[...]
    pf = p_ref[...]
    px = pf[0:1, :]
    py = pf[1:2, :]
    pz = pf[2:3, :]
    qp = jnp.dot(qf.astype(jnp.bfloat16), pf.astype(jnp.bfloat16),
                 preferred_element_type=jnp.float32)
    sqq = jnp.sum(qf * qf, axis=1, keepdims=True)
    sqp = px * px + py * py + pz * pz
    d2_ref[...] = sqq - 2.0 * qp + sqp
    big = jnp.int32(2 ** 30)
    lane = lax.broadcasted_iota(jnp.int32, (_QB, C), 1)

    def step(t, idxacc):
        d2 = d2_ref[...]
        ii = lax.broadcasted_iota(jnp.int32, (_QB, N), 1)
        m = jnp.min(d2, axis=1, keepdims=True)
        cand = jnp.where(d2 <= m, ii, big)
        j = jnp.min(cand, axis=1, keepdims=True)
        d2_ref[...] = jnp.where(cand == j, jnp.float32(jnp.inf), d2)
        return jnp.where(lane == t, j, idxacc)

    idxacc = lax.fori_loop(0, NS, step, jnp.zeros((_QB, C), jnp.int32))
    idx_ref[...] = idxacc[:, :NS]


def _knn(pts_pad, pts8):
    return pl.pallas_call(
        _knn_body,
        grid=(N // _QB,),
        in_specs=[pl.BlockSpec((_QB, 8), lambda i: (i, 0)),
                  pl.BlockSpec((8, N), lambda i: (0, 0))],
        out_specs=pl.BlockSpec((_QB, NS), lambda i: (i, 0)),
        out_shape=jax.ShapeDtypeStruct((N, NS), jnp.int32),
        scratch_shapes=[pltpu.VMEM((_QB, N), jnp.float32)],
    )(pts_pad, pts8)


_NC = 2
_NSUB = 16
_NW = _NC * _NSUB
_BPW = NT // _NW
_CH = 128
_NCH = 40
_TAIL = _BPW - _CH


def _sc_gather(k, v, pts128, idx_flat):
    mesh = plsc.VectorSubcoreMesh(core_axis_name="c", subcore_axis_name="s")

    @functools.partial(
        pl.kernel,
        mesh=mesh,
        out_type=[jax.ShapeDtypeStruct((NT, C), jnp.float32),
                  jax.ShapeDtypeStruct((NT, C), jnp.float32),
                  jax.ShapeDtypeStruct((NT, C), jnp.float32)],
        scratch_types=[pltpu.VMEM((_CH,), jnp.int32),
                       pltpu.VMEM((_CH, C), jnp.float32),
                       pltpu.VMEM((_CH, C), jnp.float32),
                       pltpu.VMEM((_CH, C), jnp.float32),
                       pltpu.SemaphoreType.DMA],
    )
    def body(k_hbm, v_hbm, p_hbm, idx_hbm, gk_hbm, gv_hbm, gp_hbm,
             idx_v, kbuf, vbuf, pbuf, sem):
        wid = lax.axis_index("s") * _NC + lax.axis_index("c")
        base = wid * _BPW

        def step(c, carry):
            off = base + jnp.minimum(c * _CH, _TAIL)
            pltpu.sync_copy(idx_hbm.at[pl.ds(off, _CH)], idx_v)
            h1 = pltpu.async_copy(k_hbm.at[idx_v], kbuf, sem)
            h2 = pltpu.async_copy(v_hbm.at[idx_v], vbuf, sem)
            h3 = pltpu.async_copy(p_hbm.at[idx_v], pbuf, sem)
            h1.wait()
            h2.wait()
            h3.wait()
            pltpu.sync_copy(kbuf, gk_hbm.at[pl.ds(off, _CH)])
            pltpu.sync_copy(vbuf, gv_hbm.at[pl.ds(off, _CH)])
            pltpu.sync_copy(pbuf, gp_hbm.at[pl.ds(off, _CH)])
            return carry

        lax.fori_loop(0, _NCH, step, 0)

    return body(k, v, pts128, idx_flat)


_RB = 200
_RTB = _RB * NS


def _rel8(gp_ref, pts_ref):
    ptsr = jnp.broadcast_to(pts_ref[...][:, None, :], (_RB, NS, 8)).reshape(_RTB, 8)
    return gp_ref[...][:, 0:8] - ptsr


def _mom_body(gp_ref, pts_ref, m1_ref, m2_ref):
    i = pl.program_id(0)
    rel = _rel8(gp_ref, pts_ref)
    m1 = jnp.sum(rel, axis=0, keepdims=True)
    m2 = lax.dot_general(rel, rel, (((0,), (0,)), ((), ())),
                         preferred_element_type=jnp.float32)

    @pl.when(i == 0)
    def _():
        m1_ref[...] = jnp.zeros_like(m1_ref)
        m2_ref[...] = jnp.zeros_like(m2_ref)

    m1_ref[...] += m1
    m2_ref[...] += m2


def _mom(gp, pts_pad):
    return pl.pallas_call(
        _mom_body,
        grid=(N // _RB,),
        in_specs=[pl.BlockSpec((_RTB, C), lambda i: (i, 0)),
                  pl.BlockSpec((_RB, 8), lambda i: (i, 0))],
        out_specs=[pl.BlockSpec((1, 8), lambda i: (0, 0)),
                   pl.BlockSpec((8, 8), lambda i: (0, 0))],
        out_shape=[jax.ShapeDtypeStruct((1, 8), jnp.float32),
                   jax.ShapeDtypeStruct((8, 8), jnp.float32)],
    )(gp, pts_pad)


def _passB_body(gp_ref, pts_ref, q_ref, gk_ref, wp_ref, bp_ref, s1_ref, t1_ref,
                wg1_ref, bg1_ref, a1_ref, st_ref):
    i = pl.program_id(0)
    rel = _rel8(gp_ref, pts_ref)
    pre = jnp.dot(rel, wp_ref[...], preferred_element_type=jnp.float32) + bp_ref[...]
    p = jnp.maximum(pre * s1_ref[...] + t1_ref[...], 0.0)
    x3 = q_ref[...] - gk_ref[...] + p.reshape(_RB, NS, C)
    a1 = jnp.dot(x3.reshape(_RTB, C), wg1_ref[...],
                 preferred_element_type=jnp.float32) + bg1_ref[...]
    a1_ref[...] = a1

    @pl.when(i == 0)
    def _():
        st_ref[...] = jnp.zeros_like(st_ref)

    st_ref[0:1, :] += jnp.sum(a1, axis=0, keepdims=True)
    st_ref[1:2, :] += jnp.sum(a1 * a1, axis=0, keepdims=True)


def _passB(gp, pts_pad, q3, gk3, wp, bp, s1, t1, wg1, bg1):
    return pl.pallas_call(
        _passB_body,
        grid=(N // _RB,),
        in_specs=[pl.BlockSpec((_RTB, C), lambda i: (i, 0)),
                  pl.BlockSpec((_RB, 8), lambda i: (i, 0)),
                  pl.BlockSpec((_RB, 1, C), lambda i: (i, 0, 0)),
                  pl.BlockSpec((_RB, NS, C), lambda i: (i, 0, 0)),
                  pl.BlockSpec((8, C), lambda i: (0, 0)),
                  pl.BlockSpec((1, C), lambda i: (0, 0)),
                  pl.BlockSpec((1, C), lambda i: (0, 0)),
                  pl.BlockSpec((1, C), lambda i: (0, 0)),
                  pl.BlockSpec((C, C), lambda i: (0, 0)),
                  pl.BlockSpec((1, C), lambda i: (0, 0))],
        out_specs=[pl.BlockSpec((_RTB, C), lambda i: (i, 0)),
                   pl.BlockSpec((2, C), lambda i: (0, 0))],
        out_shape=[jax.ShapeDtypeStruct((NT, C), jnp.float32),
                   jax.ShapeDtypeStruct((2, C), jnp.float32)],
    )(gp, pts_pad, q3, gk3, wp, bp, s1, t1, wg1, bg1)


def _passC_body(gp_ref, pts_ref, gv_ref, a1_ref, wp_ref, bp_ref, s1_ref, t1_ref,
                s2_ref, t2_ref, wg2_ref, bg2_ref, wo_ref, bo_ref, out_ref):
    rel = _rel8(gp_ref, pts_ref)
    pre = jnp.dot(rel, wp_ref[...], preferred_element_type=jnp.float32) + bp_ref[...]
    p = jnp.maximum(pre * s1_ref[...] + t1_ref[...], 0.0)
    h = jnp.maximum(a1_ref[...] * s2_ref[...] + t2_ref[...], 0.0)
    a = jnp.dot(h, wg2_ref[...], preferred_element_type=jnp.float32) + bg2_ref[...]
    a3 = a.reshape(_RB, NS, C)
    mx = jnp.max(a3, axis=1, keepdims=True)
    e = jnp.exp(a3 - mx)
    sm = e / jnp.sum(e, axis=1, keepdims=True)
    gvp = gv_ref[...] + p.reshape(_RB, NS, C)
    o = jnp.sum(gvp * sm, axis=1)
    out_ref[...] = jnp.dot(o, wo_ref[...], preferred_element_type=jnp.float32) + bo_ref[...]


def _passC(gp, pts_pad, gv3, a1, wp, bp, s1, t1, s2, t2, wg2, bg2, wo, bo):
    return pl.pallas_call(
        _passC_body,
        grid=(N // _RB,),
        in_specs=[pl.BlockSpec((_RTB, C), lambda i: (i, 0)),
                  pl.BlockSpec((_RB, 8), lambda i: (i, 0)),
                  pl.BlockSpec((_RB, NS, C), lambda i: (i, 0, 0)),
                  pl.BlockSpec((_RTB, C), lambda i: (i, 0)),
                  pl.BlockSpec((8, C), lambda i: (0, 0)),
                  pl.BlockSpec((1, C), lambda i: (0, 0)),
                  pl.BlockSpec((1, C), lambda i: (0, 0)),
                  pl.BlockSpec((1, C), lambda i: (0, 0)),
                  pl.BlockSpec((1, C), lambda i: (0, 0)),
                  pl.BlockSpec((1, C), lambda i: (0, 0)),
                  pl.BlockSpec((C, C), lambda i: (0, 0)),
                  pl.BlockSpec((1, C), lambda i: (0, 0)),
                  pl.BlockSpec((C, C), lambda i: (0, 0)),
                  pl.BlockSpec((1, C), lambda i: (0, 0))],
        out_specs=pl.BlockSpec((_RB, C), lambda i: (i, 0)),
        out_shape=jax.ShapeDtypeStruct((N, C), jnp.float32),
    )(gp, pts_pad, gv3, a1, wp, bp, s1, t1, s2, t2, wg2, bg2, wo, bo)


def kernel(points, features, Wq, bq, Wk, bk, Wv, bv, Wp, bp, g_p, be_p,
           Wg1, bg1, g_g, be_g, Wg2, bg2, Wo, bo):
    pts_pad = jnp.pad(points, ((0, 0), (0, 5)))
    pts8 = jnp.pad(points.T, ((0, 5), (0, 0)))
    pts128 = jnp.pad(points, ((0, 0), (0, C - 3)))
    wqkv = jnp.concatenate([Wq, Wk, Wv], axis=1)
    bqkv = jnp.concatenate([bq, bk, bv])[None, :]
    wp8 = jnp.pad(Wp, ((0, 5), (0, 0)))

    q, k, v = _qkv(features, wqkv, bqkv)
    idx = _knn(pts_pad, pts8)
    gk, gv, gp = _sc_gather(k, v, pts128, idx.reshape(NT))

    gk3 = gk.reshape(N, NS, C)
    gv3 = gv.reshape(N, NS, C)
    q3 = q.reshape(N, 1, C)

    m1, m2 = _mom(gp, pts_pad)

    inv = jnp.float32(1.0 / NT)
    a_mean = (m1[0] * inv) @ wp8
    quad = jnp.sum(wp8 * (m2 @ wp8), axis=0) * inv
    mean_p = a_mean + bp
    var_p = quad - a_mean * a_mean
    scale1 = g_p * lax.rsqrt(var_p + 1e-5)
    shift1 = be_p - mean_p * scale1

    a1, st = _passB(gp, pts_pad, q3, gk3, wp8, bp[None, :],
                    scale1[None, :], shift1[None, :], Wg1, bg1[None, :])

    mu_g = st[0] * inv
    var_g = st[1] * inv - mu_g * mu_g
    scale2 = g_g * lax.rsqrt(var_g + 1e-5)
    shift2 = be_g - mu_g * scale2

    return _passC(gp, pts_pad, gv3, a1, wp8, bp[None, :],
                  scale1[None, :], shift1[None, :],
                  scale2[None, :], shift2[None, :],
                  Wg2, bg2[None, :], Wo, bo[None, :])


# speedup vs baseline: 2.7824x; 1.1073x over previous
import functools

import jax
import jax.numpy as jnp
from jax import lax
from jax.experimental import pallas as pl
from jax.experimental.pallas import tpu as pltpu
from jax.experimental.pallas import tpu_sc as plsc

N = 10000
C = 128
NS = 16
NT = N * NS


_RQ = 2000


def _qkv_body(f_ref, w_ref, b_ref, q_ref, k_ref, v_ref):
    y = jnp.dot(f_ref[...], w_ref[...], preferred_element_type=jnp.float32) + b_ref[...]
    q_ref[...] = y[:, :C]
    k_ref[...] = y[:, C:2 * C]
    v_ref[...] = y[:, 2 * C:]


def _qkv(features, wqkv, bqkv):
    return pl.pallas_call(
        _qkv_body,
        grid=(N // _RQ,),
        in_specs=[pl.BlockSpec((_RQ, C), lambda i: (i, 0)),
                  pl.BlockSpec((C, 3 * C), lambda i: (0, 0)),
                  pl.BlockSpec((1, 3 * C), lambda i: (0, 0))],
        out_specs=[pl.BlockSpec((_RQ, C), lambda i: (i, 0))] * 3,
        out_shape=[jax.ShapeDtypeStruct((N, C), jnp.float32)] * 3,
    )(features, wqkv, bqkv)


_QB = 200


def _knn_body(q_ref, p_ref, idx_ref, d2_ref, ii_ref):
    qf = q_ref[...]
    pf = p_ref[...]
    px = pf[0:1, :]
    py = pf[1:2, :]
    pz = pf[2:3, :]
    qp = jnp.dot(qf.astype(jnp.bfloat16), pf.astype(jnp.bfloat16),
                 preferred_element_type=jnp.float32)
    sqq = jnp.sum(qf * qf, axis=1, keepdims=True)
    sqp = px * px + py * py + pz * pz
    d2_ref[...] = sqq - 2.0 * qp + sqp
    ii_ref[...] = lax.broadcasted_iota(jnp.int32, (_QB, N), 1)
    big = jnp.int32(2 ** 30)
    lane = lax.broadcasted_iota(jnp.int32, (_QB, C), 1)

    def step(t, idxacc):
        d2 = d2_ref[...]
        ii = ii_ref[...]
        m = jnp.min(d2, axis=1, keepdims=True)
        cand = jnp.where(d2 <= m, ii, big)
        j = jnp.min(cand, axis=1, keepdims=True)
        d2_ref[...] = jnp.where(cand == j, jnp.float32(jnp.inf), d2)
        return jnp.where(lane == t, j, idxacc)

    idxacc = lax.fori_loop(0, NS, step, jnp.zeros((_QB, C), jnp.int32))
    idx_ref[...] = idxacc[:, :NS]


def _knn(pts_pad, pts8):
    return pl.pallas_call(
        _knn_body,
        grid=(N // _QB,),
        in_specs=[pl.BlockSpec((_QB, 8), lambda i: (i, 0)),
                  pl.BlockSpec((8, N), lambda i: (0, 0))],
        out_specs=pl.BlockSpec((_QB, NS), lambda i: (i, 0)),
        out_shape=jax.ShapeDtypeStruct((N, NS), jnp.int32),
        scratch_shapes=[pltpu.VMEM((_QB, N), jnp.float32),
                        pltpu.VMEM((_QB, N), jnp.int32)],
    )(pts_pad, pts8)


_NC = 2
_NSUB = 16
_NW = _NC * _NSUB
_BPW = NT // _NW
_CH = 128
_NCH = 40
_TAIL = _BPW - _CH


def _sc_gather(k, v, pts128, idx_flat):
    mesh = plsc.VectorSubcoreMesh(core_axis_name="c", subcore_axis_name="s")

    @functools.partial(
        pl.kernel,
        mesh=mesh,
        out_type=[jax.ShapeDtypeStruct((NT, C), jnp.float32),
                  jax.ShapeDtypeStruct((NT, C), jnp.float32),
                  jax.ShapeDtypeStruct((NT, C), jnp.float32)],
        scratch_types=[pltpu.VMEM((_CH,), jnp.int32),
                       pltpu.VMEM((_CH, C), jnp.float32),
                       pltpu.VMEM((_CH, C), jnp.float32),
                       pltpu.VMEM((_CH, C), jnp.float32),
                       pltpu.SemaphoreType.DMA],
    )
    def body(k_hbm, v_hbm, p_hbm, idx_hbm, gk_hbm, gv_hbm, gp_hbm,
             idx_v, kbuf, vbuf, pbuf, sem):
        wid = lax.axis_index("s") * _NC + lax.axis_index("c")
        base = wid * _BPW

        def step(c, carry):
            off = base + jnp.minimum(c * _CH, _TAIL)
            pltpu.sync_copy(idx_hbm.at[pl.ds(off, _CH)], idx_v)
            h1 = pltpu.async_copy(k_hbm.at[idx_v], kbuf, sem)
            h2 = pltpu.async_copy(v_hbm.at[idx_v], vbuf, sem)
            h3 = pltpu.async_copy(p_hbm.at[idx_v], pbuf, sem)
            h1.wait()
            h2.wait()
            h3.wait()
            pltpu.sync_copy(kbuf, gk_hbm.at[pl.ds(off, _CH)])
            pltpu.sync_copy(vbuf, gv_hbm.at[pl.ds(off, _CH)])
            pltpu.sync_copy(pbuf, gp_hbm.at[pl.ds(off, _CH)])
            return carry

        lax.fori_loop(0, _NCH, step, 0)

    return body(k, v, pts128, idx_flat)


_RB = 200
_RTB = _RB * NS


def _rel8(gp_ref, pts_ref):
    ptsr = jnp.broadcast_to(pts_ref[...][:, None, :], (_RB, NS, 8)).reshape(_RTB, 8)
    return gp_ref[...][:, 0:8] - ptsr


def _mom_body(gp_ref, pts_ref, m1_ref, m2_ref):
    i = pl.program_id(0)
    rel = _rel8(gp_ref, pts_ref)
    m1 = jnp.sum(rel, axis=0, keepdims=True)
    m2 = lax.dot_general(rel, rel, (((0,), (0,)), ((), ())),
                         preferred_element_type=jnp.float32)

    @pl.when(i == 0)
    def _():
        m1_ref[...] = jnp.zeros_like(m1_ref)
        m2_ref[...] = jnp.zeros_like(m2_ref)

    m1_ref[...] += m1
    m2_ref[...] += m2


def _mom(gp, pts_pad):
    return pl.pallas_call(
        _mom_body,
        grid=(N // _RB,),
        in_specs=[pl.BlockSpec((_RTB, C), lambda i: (i, 0)),
                  pl.BlockSpec((_RB, 8), lambda i: (i, 0))],
        out_specs=[pl.BlockSpec((1, 8), lambda i: (0, 0)),
                   pl.BlockSpec((8, 8), lambda i: (0, 0))],
        out_shape=[jax.ShapeDtypeStruct((1, 8), jnp.float32),
                   jax.ShapeDtypeStruct((8, 8), jnp.float32)],
    )(gp, pts_pad)


def _passB_body(gp_ref, pts_ref, q_ref, gk_ref, wp_ref, bp_ref, s1_ref, t1_ref,
                wg1_ref, bg1_ref, a1_ref, st_ref):
    i = pl.program_id(0)
    rel = _rel8(gp_ref, pts_ref)
    pre = jnp.dot(rel, wp_ref[...], preferred_element_type=jnp.float32) + bp_ref[...]
    p = jnp.maximum(pre * s1_ref[...] + t1_ref[...], 0.0)
    x3 = q_ref[...] - gk_ref[...] + p.reshape(_RB, NS, C)
    a1 = jnp.dot(x3.reshape(_RTB, C), wg1_ref[...],
                 preferred_element_type=jnp.float32) + bg1_ref[...]
    a1_ref[...] = a1

    @pl.when(i == 0)
    def _():
        st_ref[...] = jnp.zeros_like(st_ref)

    st_ref[0:1, :] += jnp.sum(a1, axis=0, keepdims=True)
    st_ref[1:2, :] += jnp.sum(a1 * a1, axis=0, keepdims=True)


def _passB(gp, pts_pad, q3, gk3, wp, bp, s1, t1, wg1, bg1):
    return pl.pallas_call(
        _passB_body,
        grid=(N // _RB,),
        in_specs=[pl.BlockSpec((_RTB, C), lambda i: (i, 0)),
                  pl.BlockSpec((_RB, 8), lambda i: (i, 0)),
                  pl.BlockSpec((_RB, 1, C), lambda i: (i, 0, 0)),
                  pl.BlockSpec((_RB, NS, C), lambda i: (i, 0, 0)),
                  pl.BlockSpec((8, C), lambda i: (0, 0)),
                  pl.BlockSpec((1, C), lambda i: (0, 0)),
                  pl.BlockSpec((1, C), lambda i: (0, 0)),
                  pl.BlockSpec((1, C), lambda i: (0, 0)),
                  pl.BlockSpec((C, C), lambda i: (0, 0)),
                  pl.BlockSpec((1, C), lambda i: (0, 0))],
        out_specs=[pl.BlockSpec((_RTB, C), lambda i: (i, 0)),
                   pl.BlockSpec((2, C), lambda i: (0, 0))],
        out_shape=[jax.ShapeDtypeStruct((NT, C), jnp.float32),
                   jax.ShapeDtypeStruct((2, C), jnp.float32)],
    )(gp, pts_pad, q3, gk3, wp, bp, s1, t1, wg1, bg1)


def _passC_body(gp_ref, pts_ref, gv_ref, a1_ref, wp_ref, bp_ref, s1_ref, t1_ref,
                s2_ref, t2_ref, wg2_ref, bg2_ref, wo_ref, bo_ref, out_ref):
    rel = _rel8(gp_ref, pts_ref)
    pre = jnp.dot(rel, wp_ref[...], preferred_element_type=jnp.float32) + bp_ref[...]
    p = jnp.maximum(pre * s1_ref[...] + t1_ref[...], 0.0)
    h = jnp.maximum(a1_ref[...] * s2_ref[...] + t2_ref[...], 0.0)
    a = jnp.dot(h, wg2_ref[...], preferred_element_type=jnp.float32) + bg2_ref[...]
    a3 = a.reshape(_RB, NS, C)
    mx = jnp.max(a3, axis=1, keepdims=True)
    e = jnp.exp(a3 - mx)
    sm = e / jnp.sum(e, axis=1, keepdims=True)
    gvp = gv_ref[...] + p.reshape(_RB, NS, C)
    o = jnp.sum(gvp * sm, axis=1)
    out_ref[...] = jnp.dot(o, wo_ref[...], preferred_element_type=jnp.float32) + bo_ref[...]


def _passC(gp, pts_pad, gv3, a1, wp, bp, s1, t1, s2, t2, wg2, bg2, wo, bo):
    return pl.pallas_call(
        _passC_body,
        grid=(N // _RB,),
        in_specs=[pl.BlockSpec((_RTB, C), lambda i: (i, 0)),
                  pl.BlockSpec((_RB, 8), lambda i: (i, 0)),
                  pl.BlockSpec((_RB, NS, C), lambda i: (i, 0, 0)),
                  pl.BlockSpec((_RTB, C), lambda i: (i, 0)),
                  pl.BlockSpec((8, C), lambda i: (0, 0)),
                  pl.BlockSpec((1, C), lambda i: (0, 0)),
                  pl.BlockSpec((1, C), lambda i: (0, 0)),
                  pl.BlockSpec((1, C), lambda i: (0, 0)),
                  pl.BlockSpec((1, C), lambda i: (0, 0)),
                  pl.BlockSpec((1, C), lambda i: (0, 0)),
                  pl.BlockSpec((C, C), lambda i: (0, 0)),
                  pl.BlockSpec((1, C), lambda i: (0, 0)),
                  pl.BlockSpec((C, C), lambda i: (0, 0)),
                  pl.BlockSpec((1, C), lambda i: (0, 0))],
        out_specs=pl.BlockSpec((_RB, C), lambda i: (i, 0)),
        out_shape=jax.ShapeDtypeStruct((N, C), jnp.float32),
    )(gp, pts_pad, gv3, a1, wp, bp, s1, t1, s2, t2, wg2, bg2, wo, bo)


def kernel(points, features, Wq, bq, Wk, bk, Wv, bv, Wp, bp, g_p, be_p,
           Wg1, bg1, g_g, be_g, Wg2, bg2, Wo, bo):
    pts_pad = jnp.pad(points, ((0, 0), (0, 5)))
    pts8 = jnp.pad(points.T, ((0, 5), (0, 0)))
    pts128 = jnp.pad(points, ((0, 0), (0, C - 3)))
    wqkv = jnp.concatenate([Wq, Wk, Wv], axis=1)
    bqkv = jnp.concatenate([bq, bk, bv])[None, :]
    wp8 = jnp.pad(Wp, ((0, 5), (0, 0)))

    q, k, v = _qkv(features, wqkv, bqkv)
    idx = _knn(pts_pad, pts8)
    gk, gv, gp = _sc_gather(k, v, pts128, idx.reshape(NT))

    gk3 = gk.reshape(N, NS, C)
    gv3 = gv.reshape(N, NS, C)
    q3 = q.reshape(N, 1, C)

    m1, m2 = _mom(gp, pts_pad)

    inv = jnp.float32(1.0 / NT)
    a_mean = (m1[0] * inv) @ wp8
    quad = jnp.sum(wp8 * (m2 @ wp8), axis=0) * inv
    mean_p = a_mean + bp
    var_p = quad - a_mean * a_mean
    scale1 = g_p * lax.rsqrt(var_p + 1e-5)
    shift1 = be_p - mean_p * scale1

    a1, st = _passB(gp, pts_pad, q3, gk3, wp8, bp[None, :],
                    scale1[None, :], shift1[None, :], Wg1, bg1[None, :])

    mu_g = st[0] * inv
    var_g = st[1] * inv - mu_g * mu_g
    scale2 = g_g * lax.rsqrt(var_g + 1e-5)
    shift2 = be_g - mu_g * scale2

    return _passC(gp, pts_pad, gv3, a1, wp8, bp[None, :],
                  scale1[None, :], shift1[None, :],
                  scale2[None, :], shift2[None, :],
                  Wg2, bg2[None, :], Wo, bo[None, :])
